# baseline (speedup 1.0000x reference)
import functools

import jax
import jax.numpy as jnp
from jax import lax
from jax.experimental import pallas as pl
from jax.experimental.pallas import tpu as pltpu
from jax.experimental.pallas import tpu_sc as plsc

_NC = 2
_NS = 16
_R = 16


def _make_sc_kernel(B, T, D):
    NW = _NC * _NS
    rows_per_w = T // NW
    chunk = _R * D
    n_chunks = rows_per_w // _R

    mesh = plsc.VectorSubcoreMesh(
        core_axis_name="c", subcore_axis_name="s", num_cores=_NC
    )

    @functools.partial(
        pl.kernel,
        out_type=jax.ShapeDtypeStruct((B * T * D,), jnp.float32),
        mesh=mesh,
        scratch_types=[
            pltpu.VMEM((chunk,), jnp.float32),
            pltpu.VMEM((chunk,), jnp.float32),
        ],
    )
    def sc_kernel(x_hbm, emb_hbm, out_hbm, emb_v, x_v):
        wid = lax.axis_index("s") * _NC + lax.axis_index("c")
        t0 = wid * rows_per_w

        def chunk_body(ci, _):
            row = t0 + ci * _R
            pltpu.sync_copy(emb_hbm.at[pl.ds(row * D, chunk)], emb_v)

            def batch_body(b, _):
                off = b * (T * D) + row * D
                pltpu.sync_copy(x_hbm.at[pl.ds(off, chunk)], x_v)

                def add_body(i, _):
                    sl = pl.ds(i * 16, 16)
                    x_v[sl] = x_v[sl] + emb_v[sl]
                    return 0

                lax.fori_loop(0, chunk // 16, add_body, 0)
                pltpu.sync_copy(x_v, out_hbm.at[pl.ds(off, chunk)])
                return 0

            lax.fori_loop(0, B, batch_body, 0)
            return 0

        lax.fori_loop(0, n_chunks, chunk_body, 0)

    return sc_kernel


def kernel(x, embed_weight):
    B, T, D = x.shape
    sc = _make_sc_kernel(B, T, D)
    x_flat = x.reshape(B * T * D)
    emb_flat = embed_weight.reshape(embed_weight.shape[0] * D)
    out = sc(x_flat, emb_flat)
    return out.reshape(B, T, D)


# speedup vs baseline: 1.0791x; 1.0791x over previous
import functools

import jax
import jax.numpy as jnp
from jax import lax
from jax.experimental import pallas as pl
from jax.experimental.pallas import tpu as pltpu
from jax.experimental.pallas import tpu_sc as plsc

_NC = 2
_NS = 16
_R = 16


def _make_sc_kernel(B, T, D):
    NW = _NC * _NS
    rows_per_w = T // NW
    chunk = _R * D
    n_chunks = rows_per_w // _R

    mesh = plsc.VectorSubcoreMesh(
        core_axis_name="c", subcore_axis_name="s", num_cores=_NC
    )

    @functools.partial(
        pl.kernel,
        out_type=jax.ShapeDtypeStruct((B * T * D,), jnp.float32),
        mesh=mesh,
        scratch_types=[
            pltpu.VMEM((chunk,), jnp.float32),
            pltpu.VMEM((chunk,), jnp.float32),
            pltpu.VMEM((chunk,), jnp.float32),
            pltpu.SemaphoreType.DMA,
            pltpu.SemaphoreType.DMA,
            pltpu.SemaphoreType.DMA,
            pltpu.SemaphoreType.DMA,
        ],
    )
    def sc_kernel(x_hbm, emb_hbm, out_hbm, emb_v, xb0, xb1,
                  in_s0, in_s1, out_s0, out_s1):
        wid = lax.axis_index("s") * _NC + lax.axis_index("c")
        t0 = wid * rows_per_w

        bufs = (xb0, xb1)
        in_sems = (in_s0, in_s1)
        out_sems = (out_s0, out_s1)
        pending_out = [None, None]

        def add_chunk(buf):
            def body(i, _):
                for u in range(8):
                    sl = pl.ds((i * 8 + u) * 16, 16)
                    buf[sl] = buf[sl] + emb_v[sl]
                return 0

            lax.fori_loop(0, chunk // (16 * 8), body, 0, unroll=2)

        for ci in range(n_chunks):
            row = t0 + ci * _R
            pltpu.sync_copy(emb_hbm.at[pl.ds(row * D, chunk)], emb_v)

            if pending_out[0] is not None:
                pending_out[0].wait()
                pending_out[0] = None
            off0 = row * D
            pending_in = [None, None]
            pending_in[0] = pltpu.async_copy(
                x_hbm.at[pl.ds(off0, chunk)], bufs[0], in_sems[0])

            for b in range(B):
                cur = b % 2
                nxt = (b + 1) % 2
                if b + 1 < B:
                    if pending_out[nxt] is not None:
                        pending_out[nxt].wait()
                        pending_out[nxt] = None
                    off = (b + 1) * (T * D) + row * D
                    pending_in[nxt] = pltpu.async_copy(
                        x_hbm.at[pl.ds(off, chunk)], bufs[nxt], in_sems[nxt])
                pending_in[cur].wait()
                add_chunk(bufs[cur])
                off = b * (T * D) + row * D
                pending_out[cur] = pltpu.async_copy(
                    bufs[cur], out_hbm.at[pl.ds(off, chunk)], out_sems[cur])

        for k in range(2):
            if pending_out[k] is not None:
                pending_out[k].wait()

    return sc_kernel


def kernel(x, embed_weight):
    B, T, D = x.shape
    sc = _make_sc_kernel(B, T, D)
    x_flat = x.reshape(B * T * D)
    emb_flat = embed_weight.reshape(embed_weight.shape[0] * D)
    out = sc(x_flat, emb_flat)
    return out.reshape(B, T, D)


# speedup vs baseline: 8.4247x; 7.8073x over previous
import jax
import jax.numpy as jnp
from jax.experimental import pallas as pl


_TB = 2048


def _add_kernel(x_ref, emb_ref, out_ref):
    out_ref[...] = x_ref[...] + emb_ref[...][None, :, :]


def kernel(x, embed_weight):
    B, T, D = x.shape
    tb = min(_TB, T)
    grid = (T // tb, B)
    return pl.pallas_call(
        _add_kernel,
        grid=grid,
        in_specs=[
            pl.BlockSpec((1, tb, D), lambda i, b: (b, i, 0)),
            pl.BlockSpec((tb, D), lambda i, b: (i, 0)),
        ],
        out_specs=pl.BlockSpec((1, tb, D), lambda i, b: (b, i, 0)),
        out_shape=jax.ShapeDtypeStruct((B, T, D), x.dtype),
    )(x, embed_weight)
